# baseline (speedup 1.0000x reference)
import functools

import jax
import jax.numpy as jnp
from jax import lax
from jax.experimental import pallas as pl
from jax.experimental.pallas import tpu as pltpu
from jax.experimental.pallas import tpu_sc as plsc

EMBED = 128
LANES = 16
NUM_CORES = 2
NUM_SUBCORES = 16
NUM_WORKERS = NUM_CORES * NUM_SUBCORES
CHUNK = 128

NREL = 1000
NREL_PAD = 1024
TC_BLOCK = 2048


def _score_kernel(b_per_w, n_chunks, h_hbm, t_hbm, idx_hbm, tab_hbm, out_hbm,
                  idx_v, h_v, t_v, r_v, o_v, p_v, sems, osem):
    wid = lax.axis_index("s") * NUM_CORES + lax.axis_index("c")
    wbase = wid * b_per_w

    pltpu.sync_copy(idx_hbm.at[pl.ds(wbase, b_per_w)], idx_v)

    lane_iota = lax.iota(jnp.int32, LANES)

    def start(g):
        slot = g % 2
        base = wbase + g * CHUNK
        c_h = pltpu.async_copy(h_hbm.at[pl.ds(base, CHUNK)], h_v.at[slot],
                               sems.at[slot, 0])
        c_t = pltpu.async_copy(t_hbm.at[pl.ds(base, CHUNK)], t_v.at[slot],
                               sems.at[slot, 1])
        c_r = pltpu.async_copy(tab_hbm.at[idx_v.at[pl.ds(g * CHUNK, CHUNK)]],
                               r_v.at[slot], sems.at[slot, 2])
        return (c_h, c_t, c_r)

    pending = start(0)
    out_cps = [None, None]
    for g in range(n_chunks):
        slot = g % 2
        nxt = start(g + 1) if g + 1 < n_chunks else None
        for c in pending:
            c.wait()
        pending = nxt
        if out_cps[slot] is not None:
            out_cps[slot].wait()
        UNROLL = 8

        @plsc.parallel_loop(0, CHUNK // UNROLL, unroll=2)
        def group_body(gi):
            for rr in range(UNROLL):
                row = gi * UNROLL + rr
                accs = [None] * 4
                for jj in range(EMBED // LANES):
                    sl = pl.ds(jj * LANES, LANES)
                    p = (h_v[slot, row, sl] * r_v[slot, row, sl]
                         * t_v[slot, row, sl])
                    a = jj % 4
                    accs[a] = p if accs[a] is None else accs[a] + p
                s = (accs[0] + accs[1]) + (accs[2] + accs[3])
                plsc.store_scatter(p_v, [row * (LANES + 1) + lane_iota], s)

        @plsc.parallel_loop(0, CHUNK // LANES, unroll=1)
        def red_body(gi):
            base = lane_iota * (LANES + 1) + gi * (LANES * (LANES + 1))
            accs = [None] * 4
            for c in range(LANES):
                v = plsc.load_gather(p_v, [base + c])
                a = c % 4
                accs[a] = v if accs[a] is None else accs[a] + v
            o_v[slot, pl.ds(gi * LANES, LANES)] = (
                (accs[0] + accs[1]) + (accs[2] + accs[3]))

        out_cps[slot] = pltpu.async_copy(
            o_v.at[slot], out_hbm.at[pl.ds(wbase + g * CHUNK, CHUNK)],
            osem.at[slot])
    for c in out_cps:
        if c is not None:
            c.wait()


def _sc_half(h_emb, t_emb, idx, relation_embed, sc_rows):
    b_per_w = sc_rows // NUM_WORKERS
    n_chunks = b_per_w // CHUNK

    mesh = plsc.VectorSubcoreMesh(core_axis_name="c", subcore_axis_name="s")
    run = functools.partial(
        pl.kernel, mesh=mesh,
        compiler_params=pltpu.CompilerParams(needs_layout_passes=False),
        out_type=jax.ShapeDtypeStruct((sc_rows,), jnp.float32),
        scratch_types=[
            pltpu.VMEM((b_per_w,), jnp.int32),
            pltpu.VMEM((2, CHUNK, EMBED), jnp.float32),
            pltpu.VMEM((2, CHUNK, EMBED), jnp.float32),
            pltpu.VMEM((2, CHUNK, EMBED), jnp.float32),
            pltpu.VMEM((2, CHUNK), jnp.float32),
            pltpu.VMEM((CHUNK * (LANES + 1),), jnp.float32),
            pltpu.SemaphoreType.DMA((2, 3)),
            pltpu.SemaphoreType.DMA((2,)),
        ],
    )(functools.partial(_score_kernel, b_per_w, n_chunks))
    return run(h_emb, t_emb, idx, relation_embed)


def _tc_body(h_ref, t_ref, idx_ref, tab_ref, o_ref):
    idx = idx_ref[0, 0]
    rel_iota = lax.broadcasted_iota(jnp.int32, (TC_BLOCK, NREL_PAD), 1)
    onehot = (rel_iota == idx[:, None]).astype(jnp.bfloat16)
    r_emb = jnp.dot(onehot, tab_ref[...], preferred_element_type=jnp.float32)
    p = h_ref[...] * t_ref[...]
    o_ref[0, 0] = jnp.sum(p * r_emb, axis=1)


def _tc_half(h_emb, t_emb, idx, relation_embed, tc_start):
    batch = h_emb.shape[0]
    tc_rows = batch - tc_start
    n_blocks = tc_rows // TC_BLOCK
    first_block = tc_start // TC_BLOCK
    tab = jnp.zeros((NREL_PAD, EMBED), jnp.bfloat16).at[:NREL].set(
        relation_embed.astype(jnp.bfloat16))
    out = pl.pallas_call(
        _tc_body,
        grid=(n_blocks,),
        in_specs=[
            pl.BlockSpec((TC_BLOCK, EMBED), lambda i: (i + first_block, 0)),
            pl.BlockSpec((TC_BLOCK, EMBED), lambda i: (i + first_block, 0)),
            pl.BlockSpec((1, 1, TC_BLOCK), lambda i: (i + first_block, 0, 0)),
            pl.BlockSpec((NREL_PAD, EMBED), lambda i: (0, 0)),
        ],
        out_specs=pl.BlockSpec((1, 1, TC_BLOCK), lambda i: (i, 0, 0)),
        out_shape=jax.ShapeDtypeStruct((n_blocks, 1, TC_BLOCK), jnp.float32),
    )(h_emb, t_emb, idx.reshape(batch // TC_BLOCK, 1, TC_BLOCK), tab)
    return out.reshape(tc_rows)


def kernel(h_emb, t_emb, r_type, relation_embed):
    batch = h_emb.shape[0]
    idx = r_type.astype(jnp.int32)
    sc_rows = batch // 2
    sc_out = _sc_half(h_emb, t_emb, idx, relation_embed, sc_rows)
    tc_out = _tc_half(h_emb, t_emb, idx, relation_embed, sc_rows)
    return jnp.concatenate([sc_out, tc_out])


# speedup vs baseline: 1.0305x; 1.0305x over previous
import functools

import jax
import jax.numpy as jnp
from jax import lax
from jax.experimental import pallas as pl
from jax.experimental.pallas import tpu as pltpu
from jax.experimental.pallas import tpu_sc as plsc

EMBED = 128
LANES = 16
NUM_CORES = 2
NUM_SUBCORES = 16
NUM_WORKERS = NUM_CORES * NUM_SUBCORES
CHUNK = 128

NREL = 1000
NREL_PAD = 1024
TC_BLOCK = 2048


def _score_kernel(b_per_w, n_chunks, h_hbm, t_hbm, idx_hbm, tab_hbm, out_hbm,
                  idx_v, h_v, t_v, r_v, o_v, p_v, sems, osem):
    wid = lax.axis_index("s") * NUM_CORES + lax.axis_index("c")
    wbase = wid * b_per_w

    pltpu.sync_copy(idx_hbm.at[pl.ds(wbase, b_per_w)], idx_v)

    lane_iota = lax.iota(jnp.int32, LANES)

    def start(g):
        slot = g % 2
        base = wbase + g * CHUNK
        c_h = pltpu.async_copy(h_hbm.at[pl.ds(base, CHUNK)], h_v.at[slot],
                               sems.at[slot, 0])
        c_t = pltpu.async_copy(t_hbm.at[pl.ds(base, CHUNK)], t_v.at[slot],
                               sems.at[slot, 1])
        c_r = pltpu.async_copy(tab_hbm.at[idx_v.at[pl.ds(g * CHUNK, CHUNK)]],
                               r_v.at[slot], sems.at[slot, 2])
        return (c_h, c_t, c_r)

    pending = start(0)
    out_cps = [None, None]
    for g in range(n_chunks):
        slot = g % 2
        nxt = start(g + 1) if g + 1 < n_chunks else None
        for c in pending:
            c.wait()
        pending = nxt
        if out_cps[slot] is not None:
            out_cps[slot].wait()
        UNROLL = 8

        @plsc.parallel_loop(0, CHUNK // UNROLL, unroll=2)
        def group_body(gi):
            for rr in range(UNROLL):
                row = gi * UNROLL + rr
                accs = [None] * 4
                for jj in range(EMBED // LANES):
                    sl = pl.ds(jj * LANES, LANES)
                    p = (h_v[slot, row, sl] * r_v[slot, row, sl]
                         * t_v[slot, row, sl])
                    a = jj % 4
                    accs[a] = p if accs[a] is None else accs[a] + p
                s = (accs[0] + accs[1]) + (accs[2] + accs[3])
                plsc.store_scatter(p_v, [row * (LANES + 1) + lane_iota], s)

        @plsc.parallel_loop(0, CHUNK // LANES, unroll=1)
        def red_body(gi):
            base = lane_iota * (LANES + 1) + gi * (LANES * (LANES + 1))
            accs = [None] * 4
            for c in range(LANES):
                v = plsc.load_gather(p_v, [base + c])
                a = c % 4
                accs[a] = v if accs[a] is None else accs[a] + v
            o_v[slot, pl.ds(gi * LANES, LANES)] = (
                (accs[0] + accs[1]) + (accs[2] + accs[3]))

        out_cps[slot] = pltpu.async_copy(
            o_v.at[slot], out_hbm.at[pl.ds(wbase + g * CHUNK, CHUNK)],
            osem.at[slot])
    for c in out_cps:
        if c is not None:
            c.wait()


def _sc_half(h_emb, t_emb, idx, relation_embed, sc_rows):
    b_per_w = sc_rows // NUM_WORKERS
    n_chunks = b_per_w // CHUNK

    mesh = plsc.VectorSubcoreMesh(core_axis_name="c", subcore_axis_name="s")
    run = functools.partial(
        pl.kernel, mesh=mesh,
        compiler_params=pltpu.CompilerParams(needs_layout_passes=False),
        out_type=jax.ShapeDtypeStruct((sc_rows,), jnp.float32),
        scratch_types=[
            pltpu.VMEM((b_per_w,), jnp.int32),
            pltpu.VMEM((2, CHUNK, EMBED), jnp.float32),
            pltpu.VMEM((2, CHUNK, EMBED), jnp.float32),
            pltpu.VMEM((2, CHUNK, EMBED), jnp.float32),
            pltpu.VMEM((2, CHUNK), jnp.float32),
            pltpu.VMEM((CHUNK * (LANES + 1),), jnp.float32),
            pltpu.SemaphoreType.DMA((2, 3)),
            pltpu.SemaphoreType.DMA((2,)),
        ],
    )(functools.partial(_score_kernel, b_per_w, n_chunks))
    return run(h_emb, t_emb, idx, relation_embed)


def _tc_body(h_ref, t_ref, idx_ref, tab_ref, o_ref):
    idx = idx_ref[0, 0]
    rel_iota = lax.broadcasted_iota(jnp.int32, (TC_BLOCK, NREL_PAD), 1)
    onehot = (rel_iota == idx[:, None]).astype(jnp.bfloat16)
    r_emb = jnp.dot(onehot, tab_ref[...], preferred_element_type=jnp.float32)
    p = h_ref[...] * t_ref[...]
    o_ref[0, 0] = jnp.sum(p * r_emb, axis=1)


def _tc_half(h_emb, t_emb, idx, relation_embed, tc_start):
    batch = h_emb.shape[0]
    tc_rows = batch - tc_start
    n_blocks = tc_rows // TC_BLOCK
    first_block = tc_start // TC_BLOCK
    tab = jnp.zeros((NREL_PAD, EMBED), jnp.bfloat16).at[:NREL].set(
        relation_embed.astype(jnp.bfloat16))
    out = pl.pallas_call(
        _tc_body,
        grid=(n_blocks,),
        in_specs=[
            pl.BlockSpec((TC_BLOCK, EMBED), lambda i: (i + first_block, 0)),
            pl.BlockSpec((TC_BLOCK, EMBED), lambda i: (i + first_block, 0)),
            pl.BlockSpec((1, 1, TC_BLOCK), lambda i: (i + first_block, 0, 0)),
            pl.BlockSpec((NREL_PAD, EMBED), lambda i: (0, 0)),
        ],
        out_specs=pl.BlockSpec((1, 1, TC_BLOCK), lambda i: (i, 0, 0)),
        out_shape=jax.ShapeDtypeStruct((n_blocks, 1, TC_BLOCK), jnp.float32),
    )(h_emb, t_emb, idx.reshape(batch // TC_BLOCK, 1, TC_BLOCK), tab)
    return out.reshape(tc_rows)


def kernel(h_emb, t_emb, r_type, relation_embed):
    batch = h_emb.shape[0]
    idx = r_type.astype(jnp.int32)
    sc_rows = (batch * 3) // 4
    sc_out = _sc_half(h_emb, t_emb, idx, relation_embed, sc_rows)
    tc_out = _tc_half(h_emb, t_emb, idx, relation_embed, sc_rows)
    return jnp.concatenate([sc_out, tc_out])
